# Initial kernel scaffold; baseline (speedup 1.0000x reference)
import jax
import jax.numpy as jnp
from jax.experimental import pallas as pl


def kernel(x, edge_index, batch, w1_0, b1_0, w1_12, b1_12, w2, b2, bn_g, bn_b, bn_m, bn_v, pool_p, fc_w, fc_b):
    raise NotImplementedError("write your pallas kernel here")


# speedup vs baseline: 4.0185x; 4.0185x over previous
import functools

import jax
import jax.numpy as jnp
import numpy as np
from jax.experimental import pallas as pl
from jax.experimental.pallas import tpu as pltpu

N_NODES = 10000
RATIO = 0.8
N_LAYERS = 3
BR = 400


def _mlp_body(x_ref, agg_ref, w1_ref, b1_ref, g_ref, bb_ref, m_ref, v_ref,
              w2_ref, b2_ref, p_ref, pn_ref, h_ref, y_ref):
    h0 = x_ref[...] + agg_ref[...]
    h1 = jnp.dot(h0, w1_ref[...], preferred_element_type=jnp.float32)
    h1 = h1 + b1_ref[...]
    h1 = jnp.where(h1 > 0, h1, 0.2 * h1)
    h1 = (h1 - m_ref[...]) / jnp.sqrt(v_ref[...] + 1e-5) * g_ref[...] + bb_ref[...]
    h2 = jnp.dot(h1, w2_ref[...], preferred_element_type=jnp.float32)
    h2 = h2 + b2_ref[...]
    h2 = jnp.where(h2 > 0, h2, 0.2 * h2)
    h_ref[...] = h2
    s = jnp.dot(h2, p_ref[...], preferred_element_type=jnp.float32)
    y_ref[...] = jnp.tanh(s / pn_ref[...])


def _mlp(x, agg, w1, b1, g, bb, m, v, w2, b2, p, pnorm):
    n, f = x.shape
    h_dim = w1.shape[1]
    grid = (n // BR,)
    row = lambda i: (i, 0)
    fixed = lambda i: (0, 0)
    return pl.pallas_call(
        _mlp_body,
        grid=grid,
        in_specs=[
            pl.BlockSpec((BR, f), row),
            pl.BlockSpec((BR, f), row),
            pl.BlockSpec((f, h_dim), fixed),
            pl.BlockSpec((1, h_dim), fixed),
            pl.BlockSpec((1, h_dim), fixed),
            pl.BlockSpec((1, h_dim), fixed),
            pl.BlockSpec((1, h_dim), fixed),
            pl.BlockSpec((1, h_dim), fixed),
            pl.BlockSpec((h_dim, h_dim), fixed),
            pl.BlockSpec((1, h_dim), fixed),
            pl.BlockSpec((h_dim, 1), fixed),
            pl.BlockSpec((1, 1), fixed),
        ],
        out_specs=[
            pl.BlockSpec((BR, h_dim), row),
            pl.BlockSpec((BR, 1), row),
        ],
        out_shape=[
            jax.ShapeDtypeStruct((n, h_dim), jnp.float32),
            jax.ShapeDtypeStruct((n, 1), jnp.float32),
        ],
    )(x, agg, w1, b1.reshape(1, -1), g.reshape(1, -1), bb.reshape(1, -1),
      m.reshape(1, -1), v.reshape(1, -1), w2, b2.reshape(1, -1),
      p.reshape(-1, 1), pnorm.reshape(1, 1))


def kernel(x, edge_index, batch, w1_0, b1_0, w1_12, b1_12, w2, b2,
           bn_g, bn_b, bn_m, bn_v, pool_p, fc_w, fc_b):
    del batch
    p_norms = jnp.linalg.norm(pool_p, axis=1)
    src, dst = edge_index[0], edge_index[1]

    alive = jnp.ones((N_NODES,), dtype=bool)
    n_alive = N_NODES
    neg_ys = []
    for i in range(N_LAYERS):
        w1 = w1_0 if i == 0 else w1_12[i - 1]
        bb1 = b1_0 if i == 0 else b1_12[i - 1]
        agg = jax.ops.segment_sum(x[src], dst, num_segments=N_NODES)
        h, y2 = _mlp(x, agg, w1, bb1, bn_g[i], bn_b[i], bn_m[i], bn_v[i],
                     w2[i], b2[i], pool_p[i], p_norms[i])
        y = y2[:, 0]
        k = int(np.ceil(RATIO * n_alive))
        n_alive = k
        ym = jnp.where(alive, -y, jnp.inf)
        neg_ys.append(ym)
        order = jnp.lexsort(tuple(neg_ys))
        idx = order[:k]
        alive = jnp.zeros((N_NODES,), dtype=bool).at[idx].set(True)
        x = jnp.where(alive[:, None], h * y[:, None], 0.0)

    sums = jnp.sum(x, axis=0, keepdims=True) / n_alive
    return sums @ fc_w + fc_b


# speedup vs baseline: 25.2782x; 6.2905x over previous
import functools

import jax
import jax.numpy as jnp
import numpy as np
from jax import lax
from jax.experimental import pallas as pl
from jax.experimental.pallas import tpu as pltpu
from jax.experimental.pallas import tpu_sc as plsc

N = 10000
E = 320000
FH = 128
W = 128
NT = 16
NPAD = 10240
BR = 400

_K = [8000, 6400, 5120]


def _sc_agg_body(edge_split, x_hbm, src2_hbm, dst_hbm, zeros_hbm, out_hbm,
                 acc, sidx0, sidx1, didx0, didx1, rows0, rows1,
                 tsidx, tdidx, trows, gsem0, gsem1, zsem):
    c = lax.axis_index("c")
    s = lax.axis_index("s")
    if edge_split:
        per_tile = E // (2 * NT)
        ebase = (c * NT + s) * per_tile
    else:
        per_tile = E // NT
        ebase = s * per_tile
    nfull = per_tile // W
    tw = per_tile - nfull * W
    gbase = c * E + ebase

    @pl.when(s < 15)
    def _():
        pltpu.async_copy(zeros_hbm.at[pl.ds(s * 624, 624)],
                         acc.at[pl.ds(s * 624, 624)], zsem).wait()

    @pl.when(s == 15)
    def _():
        pltpu.async_copy(zeros_hbm.at[pl.ds(9360, 640)],
                         acc.at[pl.ds(9360, 640)], zsem).wait()

    plsc.subcore_barrier()

    def load_and_start(j, sidx, didx, rows, gsem):
        pltpu.sync_copy(src2_hbm.at[pl.ds(gbase + j * W, W)], sidx)
        pltpu.sync_copy(dst_hbm.at[pl.ds(ebase + j * W, W)], didx)
        pltpu.async_copy(x_hbm.at[sidx], rows, gsem)

    def wait_and_scatter(sidx, didx, rows, gsem):
        pltpu.make_async_copy(x_hbm.at[sidx], rows, gsem).wait()
        pltpu.sync_copy(rows, acc.at[didx], add=True)

    load_and_start(0, sidx0, didx0, rows0, gsem0)

    def pair_step(i, _):
        j0 = 2 * i
        load_and_start(j0 + 1, sidx1, didx1, rows1, gsem1)
        wait_and_scatter(sidx0, didx0, rows0, gsem0)

        @pl.when(j0 + 2 < nfull)
        def _():
            load_and_start(j0 + 2, sidx0, didx0, rows0, gsem0)

        wait_and_scatter(sidx1, didx1, rows1, gsem1)
        return ()

    lax.fori_loop(0, nfull // 2, pair_step, ())

    pltpu.sync_copy(src2_hbm.at[pl.ds(gbase + nfull * W, tw)], tsidx)
    pltpu.sync_copy(dst_hbm.at[pl.ds(ebase + nfull * W, tw)], tdidx)
    pltpu.async_copy(x_hbm.at[tsidx], trows, gsem0).wait()
    pltpu.sync_copy(trows, acc.at[tdidx], add=True)

    plsc.subcore_barrier()

    @pl.when(s < 15)
    def _():
        pltpu.async_copy(acc.at[pl.ds(s * 624, 624)],
                         out_hbm.at[pl.ds(c * N + s * 624, 624)], zsem).wait()

    @pl.when(s == 15)
    def _():
        pltpu.async_copy(acc.at[pl.ds(9360, 640)],
                         out_hbm.at[pl.ds(c * N + 9360, 640)], zsem).wait()


@functools.lru_cache(maxsize=None)
def _make_sc_agg(edge_split):
    tw = (E // (2 * NT if edge_split else NT)) % W
    mesh = plsc.VectorSubcoreMesh(core_axis_name="c", subcore_axis_name="s")
    return pl.kernel(
        functools.partial(_sc_agg_body, edge_split),
        mesh=mesh,
        out_type=jax.ShapeDtypeStruct((2 * N, FH), jnp.float32),
        scratch_types=[
            pltpu.VMEM_SHARED((N, FH), jnp.float32),
            pltpu.VMEM((W,), jnp.int32),
            pltpu.VMEM((W,), jnp.int32),
            pltpu.VMEM((W,), jnp.int32),
            pltpu.VMEM((W,), jnp.int32),
            pltpu.VMEM((W, FH), jnp.float32),
            pltpu.VMEM((W, FH), jnp.float32),
            pltpu.VMEM((tw,), jnp.int32),
            pltpu.VMEM((tw,), jnp.int32),
            pltpu.VMEM((tw, FH), jnp.float32),
            pltpu.SemaphoreType.DMA,
            pltpu.SemaphoreType.DMA,
            pltpu.SemaphoreType.DMA,
        ],
    )


def _sc_agg_l0(x, src2, dst, zeros):
    return _make_sc_agg(True)(x, src2, dst, zeros)


def _sc_agg_l12(xflat, src2, dst, zeros):
    return _make_sc_agg(False)(xflat, src2, dst, zeros)


def _mlp_body(first, x_ref, agg_ref, w1_ref, b1_ref, g_ref, bb_ref, m_ref,
              v_ref, w2_ref, b2_ref, p_ref, pn_ref, h_ref, y_ref):
    if first:
        h0 = x_ref[...] + agg_ref[0] + agg_ref[1]
    else:
        h0 = jnp.concatenate(
            [x_ref[0] + agg_ref[0], x_ref[1] + agg_ref[1]], axis=-1)
    h1 = jnp.dot(h0, w1_ref[...], preferred_element_type=jnp.float32)
    h1 = h1 + b1_ref[...]
    h1 = jnp.where(h1 > 0, h1, 0.2 * h1)
    h1 = (h1 - m_ref[...]) / jnp.sqrt(v_ref[...] + 1e-5) * g_ref[...] + bb_ref[...]
    h2 = jnp.dot(h1, w2_ref[...], preferred_element_type=jnp.float32)
    h2 = h2 + b2_ref[...]
    h2 = jnp.where(h2 > 0, h2, 0.2 * h2)
    h_ref[...] = h2
    sc = jnp.dot(h2, p_ref[...], preferred_element_type=jnp.float32)
    y_ref[...] = jnp.tanh(sc / pn_ref[...])


def _mlp(first, x, agg2, w1, b1, g, bb, m, v, w2, b2, p, pnorm):
    f_in = w1.shape[0]
    h_dim = w1.shape[1]
    grid = (N // BR,)
    row = lambda i: (i, 0)
    row3 = lambda i: (0, i, 0)
    fixed = lambda i: (0, 0)
    x_spec = (pl.BlockSpec((BR, f_in), row) if first
              else pl.BlockSpec((2, BR, FH), row3))
    return pl.pallas_call(
        functools.partial(_mlp_body, first),
        grid=grid,
        in_specs=[
            x_spec,
            pl.BlockSpec((2, BR, FH), row3),
            pl.BlockSpec((f_in, h_dim), fixed),
            pl.BlockSpec((1, h_dim), fixed),
            pl.BlockSpec((1, h_dim), fixed),
            pl.BlockSpec((1, h_dim), fixed),
            pl.BlockSpec((1, h_dim), fixed),
            pl.BlockSpec((1, h_dim), fixed),
            pl.BlockSpec((h_dim, h_dim), fixed),
            pl.BlockSpec((1, h_dim), fixed),
            pl.BlockSpec((h_dim, 1), fixed),
            pl.BlockSpec((1, 1), fixed),
        ],
        out_specs=[
            pl.BlockSpec((BR, h_dim), row),
            pl.BlockSpec((BR, 1), row),
        ],
        out_shape=[
            jax.ShapeDtypeStruct((N, h_dim), jnp.float32),
            jax.ShapeDtypeStruct((N, 1), jnp.float32),
        ],
    )(x, agg2, w1, b1.reshape(1, -1), g.reshape(1, -1), bb.reshape(1, -1),
      m.reshape(1, -1), v.reshape(1, -1), w2, b2.reshape(1, -1),
      p.reshape(-1, 1), pnorm.reshape(1, 1))


def _kth_largest(masked_key, need, nbits, t0):
    def bit_step(b, t):
        cand = t | (jnp.uint32(1) << (nbits - 1 - b).astype(jnp.uint32))
        cnt = jnp.sum((masked_key >= cand).astype(jnp.int32))
        return jnp.where(cnt >= need, cand, t)
    return lax.fori_loop(0, nbits, bit_step, t0)


def _topk_body(k, nprev, y_ref, alive_ref, *rest):
    key_refs = rest[:nprev]
    alive_out, key_out = rest[nprev], rest[nprev + 1]
    y = y_ref[...]
    alive = alive_ref[...] != 0
    ub = lax.bitcast_convert_type(y, jnp.uint32)
    key = jnp.where((ub >> 31) == 1, ~ub, ub | jnp.uint32(0x80000000))
    key = jnp.where(alive, key, jnp.uint32(0))
    key_out[...] = key

    flat = (lax.broadcasted_iota(jnp.uint32, (80, 128), 0) * 128
            + lax.broadcasted_iota(jnp.uint32, (80, 128), 1))
    idxkey = jnp.uint32(0xFFFFFFFF) - flat

    keys = [key] + [r[...] for r in key_refs]
    sel = jnp.zeros((80, 128), dtype=jnp.bool_)
    s_set = alive
    need = jnp.int32(k)
    for kk in keys:
        mk = jnp.where(s_set, kk, jnp.uint32(0))
        t = _kth_largest(mk, need, 32, jnp.uint32(0))
        gt = s_set & (kk > t)
        sel = sel | gt
        need = need - jnp.sum(gt.astype(jnp.int32))
        s_set = s_set & (kk == t)
    mk = jnp.where(s_set, idxkey, jnp.uint32(0))
    t = _kth_largest(mk, need, 14, jnp.uint32(0xFFFFC000))
    sel = sel | (s_set & (idxkey >= t))
    alive_out[...] = sel.astype(jnp.int32)


def _topk(k, ypad, alive, prev_keys):
    nprev = len(prev_keys)
    full = lambda i: (0, 0)
    return pl.pallas_call(
        functools.partial(_topk_body, k, nprev),
        grid=(1,),
        in_specs=[pl.BlockSpec((80, 128), full)] * (2 + nprev),
        out_specs=[pl.BlockSpec((80, 128), full)] * 2,
        out_shape=[
            jax.ShapeDtypeStruct((80, 128), jnp.int32),
            jax.ShapeDtypeStruct((80, 128), jnp.uint32),
        ],
    )(ypad, alive, *prev_keys)


def _xupd_body(h_ref, y_ref, a_ref, xs_ref):
    sc = y_ref[...] * a_ref[...]
    h = h_ref[...]
    xs_ref[0] = h[:, :FH] * sc
    xs_ref[1] = h[:, FH:] * sc


def _xupd(h, y, alivef):
    grid = (N // BR,)
    return pl.pallas_call(
        _xupd_body,
        grid=grid,
        in_specs=[
            pl.BlockSpec((BR, 2 * FH), lambda i: (i, 0)),
            pl.BlockSpec((BR, 1), lambda i: (i, 0)),
            pl.BlockSpec((BR, 1), lambda i: (i, 0)),
        ],
        out_specs=pl.BlockSpec((2, BR, FH), lambda i: (0, i, 0)),
        out_shape=jax.ShapeDtypeStruct((2, N, FH), jnp.float32),
    )(h, y, alivef)


def _readout_body(k, h_ref, y_ref, a_ref, w_ref, b_ref, o_ref):
    sc = y_ref[...] * a_ref[...]
    colsum = jnp.sum(h_ref[...] * sc, axis=0, keepdims=True)
    o_ref[...] = (jnp.dot(colsum / k, w_ref[...],
                          preferred_element_type=jnp.float32) + b_ref[...])


def _readout(k, h, y, alivef, fc_w, fc_b):
    full = lambda: (0, 0)
    return pl.pallas_call(
        functools.partial(_readout_body, float(k)),
        in_specs=[
            pl.BlockSpec(h.shape, None),
            pl.BlockSpec(y.shape, None),
            pl.BlockSpec(alivef.shape, None),
            pl.BlockSpec(fc_w.shape, None),
            pl.BlockSpec((1, fc_b.shape[0]), None),
        ],
        out_specs=pl.BlockSpec((1, fc_b.shape[0]), None),
        out_shape=jax.ShapeDtypeStruct((1, fc_b.shape[0]), jnp.float32),
    )(h, y, alivef, fc_w, fc_b.reshape(1, -1))


def _pad_y(y):
    return jnp.pad(y, ((0, NPAD - N), (0, 0))).reshape(80, 128)


def _alive_rows(alive):
    return alive.reshape(NPAD, 1)[:N].astype(jnp.float32)


def kernel(x, edge_index, batch, w1_0, b1_0, w1_12, b1_12, w2, b2,
           bn_g, bn_b, bn_m, bn_v, pool_p, fc_w, fc_b):
    del batch
    p_norms = jnp.linalg.norm(pool_p, axis=1)
    src, dst = edge_index[0], edge_index[1]
    src2_l0 = jnp.concatenate([src, src])
    src2_l12 = jnp.concatenate([src, src + N])
    zeros = jnp.zeros((N, FH), jnp.float32)
    alive = jnp.pad(jnp.ones((N,), jnp.int32), (0, NPAD - N)).reshape(80, 128)

    keys = []
    xs = None
    h = y = None
    for i in range(3):
        if i == 0:
            agg2 = _sc_agg_l0(x, src2_l0, dst, zeros)
        else:
            agg2 = _sc_agg_l12(xs.reshape(2 * N, FH), src2_l12, dst, zeros)
        w1 = w1_0 if i == 0 else w1_12[i - 1]
        bb1 = b1_0 if i == 0 else b1_12[i - 1]
        h, y = _mlp(i == 0, x if i == 0 else xs, agg2.reshape(2, N, FH),
                    w1, bb1, bn_g[i], bn_b[i], bn_m[i], bn_v[i],
                    w2[i], b2[i], pool_p[i], p_norms[i])
        alive, key = _topk(_K[i], _pad_y(y), alive, keys)
        keys = [key] + keys
        if i < 2:
            xs = _xupd(h, y, _alive_rows(alive))

    return _readout(_K[2], h, y, _alive_rows(alive), fc_w, fc_b)
